# baseline (speedup 1.0000x reference)
import functools

import jax
import jax.numpy as jnp
from jax import lax
from jax.experimental import pallas as pl
from jax.experimental.pallas import tpu as pltpu
from jax.experimental.pallas import tpu_sc as plsc

N = 10000
E = 160000
NW = 32
G = 128
C = 40
EPW = C * G
E_PAD = NW * EPW
N_PAD = 10112
RPT = N_PAD // 16
CW = 128

_MESH = plsc.VectorSubcoreMesh(core_axis_name="c", subcore_axis_name="s")


C_SC0 = 18
C_SC1 = 62
C_MAX = max(C_SC0, C_SC1)


def _prep_edges(ei):
    pad = E_PAD - E
    src = jnp.concatenate([ei[0], jnp.zeros((pad,), jnp.int32)])
    dst = jnp.concatenate([ei[1], jnp.full((pad,), N, jnp.int32)])
    k = 16 * C_SC0 * G
    return (src.reshape(NW, C, G), dst.reshape(NW, C, G),
            src[:k].reshape(16, C_SC0, G), dst[:k].reshape(16, C_SC0, G),
            src[k:].reshape(16, C_SC1, G), dst[k:].reshape(16, C_SC1, G))


def _counts_body(d0, d1, d2, ones_h, zeros_h, out, dst_v, ones_v, acc, sem):
    cid = lax.axis_index("c")
    sid = lax.axis_index("s")
    wid = cid * 16 + sid
    pltpu.sync_copy(ones_h, ones_v)
    for r, d in enumerate((d0, d1, d2)):
        pltpu.sync_copy(zeros_h, acc.at[pl.ds(sid * RPT, RPT)])
        pltpu.sync_copy(d.at[wid], dst_v)
        plsc.subcore_barrier()
        descs = [pltpu.async_copy(ones_v, acc.at[dst_v.at[c]], sem, add=True)
                 for c in range(C)]
        for desc in descs:
            desc.wait()
        plsc.subcore_barrier()
        pltpu.sync_copy(acc.at[pl.ds(sid * RPT, RPT)],
                        out.at[r, cid, pl.ds(sid * RPT, RPT)])


_counts_call = pl.kernel(
    _counts_body,
    out_type=jax.ShapeDtypeStruct((3, 2, N_PAD, CW), jnp.float32),
    mesh=_MESH,
    scratch_types=[
        pltpu.VMEM((C, G), jnp.int32),
        pltpu.VMEM((G, CW), jnp.float32),
        pltpu.VMEM_SHARED((N_PAD, CW), jnp.float32),
        pltpu.SemaphoreType.DMA,
    ],
)


NB = 2
NR = C // NB


def _make_agg(passes, n_y, n_rel):
    def body(*refs):
        ys = refs[0:n_y]
        s0 = refs[n_y:n_y + n_rel]
        d0 = refs[n_y + n_rel:n_y + 2 * n_rel]
        s1 = refs[n_y + 2 * n_rel:n_y + 3 * n_rel]
        d1 = refs[n_y + 3 * n_rel:n_y + 4 * n_rel]
        zeros_h = refs[n_y + 4 * n_rel]
        out = refs[n_y + 4 * n_rel + 1]
        rest = refs[n_y + 4 * n_rel + 2:]
        src_v, dst_v = rest[0], rest[1]
        bufs = rest[2:2 + NB]
        gsems = rest[2 + NB:2 + 2 * NB]
        ssems = rest[2 + 2 * NB:2 + 3 * NB]
        acc = rest[2 + 3 * NB]
        cid = lax.axis_index("c")
        sid = lax.axis_index("s")
        n_rounds = jnp.where(cid == 0, C_SC0 // NB, C_SC1 // NB)
        for p, (yi, r) in enumerate(passes):
            pltpu.sync_copy(zeros_h, acc.at[pl.ds(sid * RPT, RPT)])

            @pl.when(cid == 0)
            def _():
                pltpu.sync_copy(s0[r].at[sid], src_v.at[pl.ds(0, C_SC0)])
                pltpu.sync_copy(d0[r].at[sid], dst_v.at[pl.ds(0, C_SC0)])

            @pl.when(cid == 1)
            def _():
                pltpu.sync_copy(s1[r].at[sid], src_v.at[pl.ds(0, C_SC1)])
                pltpu.sync_copy(d1[r].at[sid], dst_v.at[pl.ds(0, C_SC1)])

            plsc.subcore_barrier()
            y = ys[yi]
            for j in range(NB):
                pltpu.async_copy(y.at[src_v.at[j]], bufs[j], gsems[j])

            @pl.loop(0, n_rounds)
            def _(rnd):
                base = rnd * NB
                sds = []
                for j in range(NB):
                    pltpu.make_async_copy(
                        y.at[src_v.at[base + j]], bufs[j], gsems[j]).wait()
                    sds.append(pltpu.async_copy(
                        bufs[j], acc.at[dst_v.at[base + j]], ssems[j],
                        add=True))
                for j in range(NB):
                    sds[j].wait()

                    @pl.when(rnd < n_rounds - 1)
                    def _():
                        pltpu.async_copy(
                            y.at[src_v.at[base + NB + j]], bufs[j], gsems[j])

            plsc.subcore_barrier()
            pltpu.sync_copy(acc.at[pl.ds(sid * RPT, RPT)],
                            out.at[p, cid, pl.ds(sid * RPT, RPT)])

    return pl.kernel(
        body,
        out_type=jax.ShapeDtypeStruct((len(passes), 2, N_PAD, G), jnp.float32),
        mesh=_MESH,
        scratch_types=(
            [pltpu.VMEM((C_MAX, G), jnp.int32),
             pltpu.VMEM((C_MAX, G), jnp.int32)]
            + [pltpu.VMEM((G, G), jnp.float32) for _ in range(NB)]
            + [pltpu.SemaphoreType.DMA for _ in range(2 * NB)]
            + [pltpu.VMEM_SHARED((N_PAD, G), jnp.float32)]
        ),
    )


_agg6 = _make_agg(
    passes=[(0, 0), (1, 0), (2, 1), (3, 1), (4, 2), (5, 2)], n_y=6, n_rel=3)
_agg3 = _make_agg(passes=[(0, 0), (1, 1), (2, 2)], n_y=3, n_rel=3)


BM = 400


def _mm_kernel(x_ref, w_ref, o_ref):
    o_ref[...] = jnp.dot(x_ref[...], w_ref[...],
                         preferred_element_type=jnp.float32)


def _mm(x, w):
    m, k = x.shape
    n = w.shape[1]
    return pl.pallas_call(
        _mm_kernel,
        grid=(m // BM,),
        in_specs=[
            pl.BlockSpec((BM, k), lambda i: (i, 0)),
            pl.BlockSpec((k, n), lambda i: (0, 0)),
        ],
        out_specs=pl.BlockSpec((BM, n), lambda i: (i, 0)),
        out_shape=jax.ShapeDtypeStruct((m, n), jnp.float32),
    )(x, w)


def _post_kernel(nrel, relu, root_ref, b_ref, *refs):
    o_ref = refs[-1]
    acc = root_ref[...] + b_ref[...]
    for i in range(nrel):
        part = refs[2 * i][...]
        cnt = refs[2 * i + 1][...]
        s = part[0] + part[1]
        c = cnt[0, :, 0:1] + cnt[1, :, 0:1]
        acc = acc + s / jnp.maximum(c, 1.0)
    if relu:
        acc = jnp.maximum(acc, 0.0)
    o_ref[...] = acc


def _post(root, bias, parts_cnts, relu):
    nrel = len(parts_cnts) // 2
    in_specs = [
        pl.BlockSpec((BM, G), lambda i: (i, 0)),
        pl.BlockSpec((1, G), lambda i: (0, 0)),
    ]
    for i in range(nrel):
        in_specs.append(pl.BlockSpec((2, BM, G), lambda i: (0, i, 0)))
        in_specs.append(pl.BlockSpec((2, BM, CW), lambda i: (0, i, 0)))
    return pl.pallas_call(
        functools.partial(_post_kernel, nrel, relu),
        grid=(N // BM,),
        in_specs=in_specs,
        out_specs=pl.BlockSpec((BM, G), lambda i: (i, 0)),
        out_shape=jax.ShapeDtypeStruct((N, G), jnp.float32),
    )(root, bias.reshape(1, G), *parts_cnts)


def kernel(x_paper, emb_author, Wr1_cites, Wr1_writes, Wr1_written,
           Wroot1_paper, broot1_paper, Wroot1_author, broot1_author,
           Wr2_cites, Wr2_writes, Wr2_written, Wroot2_paper, broot2_paper,
           Wroot2_author, broot2_author, edge_index_cites, edge_index_writes,
           edge_index_written):
    src_c, dst_c, sc0_c, dc0_c, sc1_c, dc1_c = _prep_edges(edge_index_cites)
    src_w, dst_w, sc0_w, dc0_w, sc1_w, dc1_w = _prep_edges(edge_index_writes)
    src_n, dst_n, sc0_n, dc0_n, sc1_n, dc1_n = _prep_edges(edge_index_written)
    asym = (sc0_c, sc0_w, sc0_n, dc0_c, dc0_w, dc0_n,
            sc1_c, sc1_w, sc1_n, dc1_c, dc1_w, dc1_n)

    ones16 = jnp.ones((G, CW), jnp.float32)
    zeros16 = jnp.zeros((RPT, CW), jnp.float32)
    zeros128 = jnp.zeros((RPT, G), jnp.float32)

    cnts = _counts_call(dst_c, dst_w, dst_n, ones16, zeros16)
    cnt_c, cnt_w, cnt_n = cnts[0], cnts[1], cnts[2]

    yc = _mm(x_paper, Wr1_cites)
    yw = _mm(emb_author, Wr1_writes)
    yn = _mm(x_paper, Wr1_written)
    rp = _mm(x_paper, Wroot1_paper)
    ra = _mm(emb_author, Wroot1_author)

    parts = _agg6(yc[:, :G], yc[:, G:], yw[:, :G], yw[:, G:],
                  yn[:, :G], yn[:, G:], *asym, zeros128)

    hp = jnp.concatenate([
        _post(rp[:, :G], broot1_paper[:G],
              [parts[0], cnt_c, parts[2], cnt_w], True),
        _post(rp[:, G:], broot1_paper[G:],
              [parts[1], cnt_c, parts[3], cnt_w], True),
    ], axis=1)
    ha = jnp.concatenate([
        _post(ra[:, :G], broot1_author[:G], [parts[4], cnt_n], True),
        _post(ra[:, G:], broot1_author[G:], [parts[5], cnt_n], True),
    ], axis=1)

    yc2 = _mm(hp, Wr2_cites)
    yw2 = _mm(ha, Wr2_writes)
    yn2 = _mm(hp, Wr2_written)
    rp2 = _mm(hp, Wroot2_paper)
    ra2 = _mm(ha, Wroot2_author)

    parts2 = _agg3(yc2, yw2, yn2, *asym, zeros128)

    p = _post(rp2, broot2_paper,
              [parts2[0], cnt_c, parts2[1], cnt_w], False)
    a = _post(ra2, broot2_author, [parts2[2], cnt_n], False)
    return p, a


# speedup vs baseline: 1.1237x; 1.1237x over previous
import functools

import jax
import jax.numpy as jnp
from jax import lax
from jax.experimental import pallas as pl
from jax.experimental.pallas import tpu as pltpu
from jax.experimental.pallas import tpu_sc as plsc

N = 10000
E = 160000
NW = 32
G = 128
C = 40
EPW = C * G
E_PAD = NW * EPW
N_PAD = 10112
RPT = N_PAD // 16
CW = 128

_MESH = plsc.VectorSubcoreMesh(core_axis_name="c", subcore_axis_name="s")


C_SC0 = 62
C_SC1 = 18
C_MAX = max(C_SC0, C_SC1)


def _prep_edges(ei):
    pad = E_PAD - E
    src = jnp.concatenate([ei[0], jnp.zeros((pad,), jnp.int32)])
    dst = jnp.concatenate([ei[1], jnp.full((pad,), N, jnp.int32)])
    k = 16 * C_SC0 * G
    return (src.reshape(NW, C, G), dst.reshape(NW, C, G),
            src[:k].reshape(16, C_SC0, G), dst[:k].reshape(16, C_SC0, G),
            src[k:].reshape(16, C_SC1, G), dst[k:].reshape(16, C_SC1, G))


def _counts_body(d0, d1, d2, ones_h, zeros_h, out, dst_v, ones_v, acc, sem):
    cid = lax.axis_index("c")
    sid = lax.axis_index("s")
    wid = cid * 16 + sid
    pltpu.sync_copy(ones_h, ones_v)
    for r, d in enumerate((d0, d1, d2)):
        pltpu.sync_copy(zeros_h, acc.at[pl.ds(sid * RPT, RPT)])
        pltpu.sync_copy(d.at[wid], dst_v)
        plsc.subcore_barrier()
        descs = [pltpu.async_copy(ones_v, acc.at[dst_v.at[c]], sem, add=True)
                 for c in range(C)]
        for desc in descs:
            desc.wait()
        plsc.subcore_barrier()
        pltpu.sync_copy(acc.at[pl.ds(sid * RPT, RPT)],
                        out.at[r, cid, pl.ds(sid * RPT, RPT)])


_counts_call = pl.kernel(
    _counts_body,
    out_type=jax.ShapeDtypeStruct((3, 2, N_PAD, CW), jnp.float32),
    mesh=_MESH,
    scratch_types=[
        pltpu.VMEM((C, G), jnp.int32),
        pltpu.VMEM((G, CW), jnp.float32),
        pltpu.VMEM_SHARED((N_PAD, CW), jnp.float32),
        pltpu.SemaphoreType.DMA,
    ],
)


NB = 2
NR = C // NB


def _make_agg(passes, n_y, n_rel):
    def body(*refs):
        ys = refs[0:n_y]
        s0 = refs[n_y:n_y + n_rel]
        d0 = refs[n_y + n_rel:n_y + 2 * n_rel]
        s1 = refs[n_y + 2 * n_rel:n_y + 3 * n_rel]
        d1 = refs[n_y + 3 * n_rel:n_y + 4 * n_rel]
        zeros_h = refs[n_y + 4 * n_rel]
        out = refs[n_y + 4 * n_rel + 1]
        rest = refs[n_y + 4 * n_rel + 2:]
        src_v, dst_v = rest[0], rest[1]
        bufs = rest[2:2 + NB]
        gsems = rest[2 + NB:2 + 2 * NB]
        ssems = rest[2 + 2 * NB:2 + 3 * NB]
        acc = rest[2 + 3 * NB]
        cid = lax.axis_index("c")
        sid = lax.axis_index("s")
        n_rounds = jnp.where(cid == 0, C_SC0 // NB, C_SC1 // NB)
        for p, (yi, r) in enumerate(passes):
            pltpu.sync_copy(zeros_h, acc.at[pl.ds(sid * RPT, RPT)])

            @pl.when(cid == 0)
            def _():
                pltpu.sync_copy(s0[r].at[sid], src_v.at[pl.ds(0, C_SC0)])
                pltpu.sync_copy(d0[r].at[sid], dst_v.at[pl.ds(0, C_SC0)])

            @pl.when(cid == 1)
            def _():
                pltpu.sync_copy(s1[r].at[sid], src_v.at[pl.ds(0, C_SC1)])
                pltpu.sync_copy(d1[r].at[sid], dst_v.at[pl.ds(0, C_SC1)])

            plsc.subcore_barrier()
            y = ys[yi]
            for j in range(NB):
                pltpu.async_copy(y.at[src_v.at[j]], bufs[j], gsems[j])

            @pl.loop(0, n_rounds)
            def _(rnd):
                base = rnd * NB
                sds = []
                for j in range(NB):
                    pltpu.make_async_copy(
                        y.at[src_v.at[base + j]], bufs[j], gsems[j]).wait()
                    sds.append(pltpu.async_copy(
                        bufs[j], acc.at[dst_v.at[base + j]], ssems[j],
                        add=True))
                for j in range(NB):
                    sds[j].wait()

                    @pl.when(rnd < n_rounds - 1)
                    def _():
                        pltpu.async_copy(
                            y.at[src_v.at[base + NB + j]], bufs[j], gsems[j])

            plsc.subcore_barrier()
            pltpu.sync_copy(acc.at[pl.ds(sid * RPT, RPT)],
                            out.at[p, cid, pl.ds(sid * RPT, RPT)])

    return pl.kernel(
        body,
        out_type=jax.ShapeDtypeStruct((len(passes), 2, N_PAD, G), jnp.float32),
        mesh=_MESH,
        scratch_types=(
            [pltpu.VMEM((C_MAX, G), jnp.int32),
             pltpu.VMEM((C_MAX, G), jnp.int32)]
            + [pltpu.VMEM((G, G), jnp.float32) for _ in range(NB)]
            + [pltpu.SemaphoreType.DMA for _ in range(2 * NB)]
            + [pltpu.VMEM_SHARED((N_PAD, G), jnp.float32)]
        ),
    )


_agg6 = _make_agg(
    passes=[(0, 0), (1, 0), (2, 1), (3, 1), (4, 2), (5, 2)], n_y=6, n_rel=3)
_agg3 = _make_agg(passes=[(0, 0), (1, 1), (2, 2)], n_y=3, n_rel=3)


BM = 400


def _mm_kernel(x_ref, w_ref, o_ref):
    o_ref[...] = jnp.dot(x_ref[...], w_ref[...],
                         preferred_element_type=jnp.float32)


def _mm(x, w):
    m, k = x.shape
    n = w.shape[1]
    return pl.pallas_call(
        _mm_kernel,
        grid=(m // BM,),
        in_specs=[
            pl.BlockSpec((BM, k), lambda i: (i, 0)),
            pl.BlockSpec((k, n), lambda i: (0, 0)),
        ],
        out_specs=pl.BlockSpec((BM, n), lambda i: (i, 0)),
        out_shape=jax.ShapeDtypeStruct((m, n), jnp.float32),
    )(x, w)


def _post_kernel(nrel, relu, root_ref, b_ref, *refs):
    o_ref = refs[-1]
    acc = root_ref[...] + b_ref[...]
    for i in range(nrel):
        part = refs[2 * i][...]
        cnt = refs[2 * i + 1][...]
        s = part[0] + part[1]
        c = cnt[0, :, 0:1] + cnt[1, :, 0:1]
        acc = acc + s / jnp.maximum(c, 1.0)
    if relu:
        acc = jnp.maximum(acc, 0.0)
    o_ref[...] = acc


def _post(root, bias, parts_cnts, relu):
    nrel = len(parts_cnts) // 2
    in_specs = [
        pl.BlockSpec((BM, G), lambda i: (i, 0)),
        pl.BlockSpec((1, G), lambda i: (0, 0)),
    ]
    for i in range(nrel):
        in_specs.append(pl.BlockSpec((2, BM, G), lambda i: (0, i, 0)))
        in_specs.append(pl.BlockSpec((2, BM, CW), lambda i: (0, i, 0)))
    return pl.pallas_call(
        functools.partial(_post_kernel, nrel, relu),
        grid=(N // BM,),
        in_specs=in_specs,
        out_specs=pl.BlockSpec((BM, G), lambda i: (i, 0)),
        out_shape=jax.ShapeDtypeStruct((N, G), jnp.float32),
    )(root, bias.reshape(1, G), *parts_cnts)


def kernel(x_paper, emb_author, Wr1_cites, Wr1_writes, Wr1_written,
           Wroot1_paper, broot1_paper, Wroot1_author, broot1_author,
           Wr2_cites, Wr2_writes, Wr2_written, Wroot2_paper, broot2_paper,
           Wroot2_author, broot2_author, edge_index_cites, edge_index_writes,
           edge_index_written):
    src_c, dst_c, sc0_c, dc0_c, sc1_c, dc1_c = _prep_edges(edge_index_cites)
    src_w, dst_w, sc0_w, dc0_w, sc1_w, dc1_w = _prep_edges(edge_index_writes)
    src_n, dst_n, sc0_n, dc0_n, sc1_n, dc1_n = _prep_edges(edge_index_written)
    asym = (sc0_c, sc0_w, sc0_n, dc0_c, dc0_w, dc0_n,
            sc1_c, sc1_w, sc1_n, dc1_c, dc1_w, dc1_n)

    ones16 = jnp.ones((G, CW), jnp.float32)
    zeros16 = jnp.zeros((RPT, CW), jnp.float32)
    zeros128 = jnp.zeros((RPT, G), jnp.float32)

    cnts = _counts_call(dst_c, dst_w, dst_n, ones16, zeros16)
    cnt_c, cnt_w, cnt_n = cnts[0], cnts[1], cnts[2]

    yc = _mm(x_paper, Wr1_cites)
    yw = _mm(emb_author, Wr1_writes)
    yn = _mm(x_paper, Wr1_written)
    rp = _mm(x_paper, Wroot1_paper)
    ra = _mm(emb_author, Wroot1_author)

    parts = _agg6(yc[:, :G], yc[:, G:], yw[:, :G], yw[:, G:],
                  yn[:, :G], yn[:, G:], *asym, zeros128)

    hp = jnp.concatenate([
        _post(rp[:, :G], broot1_paper[:G],
              [parts[0], cnt_c, parts[2], cnt_w], True),
        _post(rp[:, G:], broot1_paper[G:],
              [parts[1], cnt_c, parts[3], cnt_w], True),
    ], axis=1)
    ha = jnp.concatenate([
        _post(ra[:, :G], broot1_author[:G], [parts[4], cnt_n], True),
        _post(ra[:, G:], broot1_author[G:], [parts[5], cnt_n], True),
    ], axis=1)

    yc2 = _mm(hp, Wr2_cites)
    yw2 = _mm(ha, Wr2_writes)
    yn2 = _mm(hp, Wr2_written)
    rp2 = _mm(hp, Wroot2_paper)
    ra2 = _mm(ha, Wroot2_author)

    parts2 = _agg3(yc2, yw2, yn2, *asym, zeros128)

    p = _post(rp2, broot2_paper,
              [parts2[0], cnt_c, parts2[1], cnt_w], False)
    a = _post(ra2, broot2_author, [parts2[2], cnt_n], False)
    return p, a
